# baseline (speedup 1.0000x reference)
import functools

import jax
import jax.numpy as jnp
from jax import lax
from jax.experimental import pallas as pl
from jax.experimental.pallas import tpu as pltpu
from jax.experimental.pallas import tpu_sc as plsc

T = 2048
D = 1024
DFF = 1024
EXP = 8
KTOP = 2
NA = T * KTOP

BT = 256
MAXBLK = NA // BT + EXP - 1
BT1 = 512

NC = 2
NS = 16
NW = NC * NS


def _route_body(x_ref, gwt_ref, logits_ref, wts_ref, topi_ref, rank_ref,
                offp_ref, bexp_ref, nact_ref, xb_ref, carry):
    i = pl.program_id(0)
    nb = pl.num_programs(0)

    @pl.when(i == 0)
    def _init():
        carry[...] = jnp.zeros_like(carry)

    x = x_ref[...]
    xb_ref[...] = x.astype(jnp.bfloat16)
    logits = jnp.dot(x, gwt_ref[...], preferred_element_type=jnp.float32)
    logits_ref[...] = logits

    m = jnp.max(logits, axis=1, keepdims=True)
    ex = jnp.exp(logits - m)
    p = ex / jnp.sum(ex, axis=1, keepdims=True)

    iota_e = lax.broadcasted_iota(jnp.int32, (BT1, EXP), 1)
    v0 = jnp.max(p, axis=1, keepdims=True)
    i0 = jnp.min(jnp.where(p == v0, iota_e, EXP), axis=1, keepdims=True)
    pm = jnp.where(iota_e == i0, -jnp.inf, p)
    v1 = jnp.max(pm, axis=1, keepdims=True)
    i1 = jnp.min(jnp.where(pm == v1, iota_e, EXP), axis=1, keepdims=True)
    s = v0 + v1
    wts_ref[...] = jnp.concatenate([v0 / s, v1 / s], axis=1)
    topi_ref[...] = jnp.concatenate([i0, i1], axis=1)

    oh0 = (iota_e == i0).astype(jnp.float32)
    oh1 = (iota_e == i1).astype(jnp.float32)
    r_i = lax.broadcasted_iota(jnp.int32, (BT1, BT1), 0)
    c_i = lax.broadcasted_iota(jnp.int32, (BT1, BT1), 1)
    ltri = (c_i < r_i).astype(jnp.float32)
    cums = jnp.dot(ltri, jnp.concatenate([oh0, oh1], axis=1),
                   preferred_element_type=jnp.float32)
    cums0 = cums[:, :EXP]
    cums1 = cums[:, EXP:]
    base = carry[...]
    r0 = jnp.sum(oh0 * (base + cums0 + cums1), axis=1, keepdims=True)
    r1 = jnp.sum(oh1 * (base + cums0 + oh0 + cums1), axis=1, keepdims=True)
    rank_ref[...] = jnp.concatenate([r0, r1], axis=1).astype(jnp.int32)
    newc = base + jnp.sum(oh0 + oh1, axis=0, keepdims=True)
    carry[...] = newc

    @pl.when(i == nb - 1)
    def _epilogue():
        g = newc
        nblk = jnp.floor((g + (BT - 1)) * (1.0 / BT))
        e_r = lax.broadcasted_iota(jnp.int32, (EXP, EXP), 0)
        e_c = lax.broadcasted_iota(jnp.int32, (EXP, EXP), 1)
        m_strict = (e_r < e_c).astype(jnp.float32)
        m_incl = (e_r <= e_c).astype(jnp.float32)
        offb = jnp.dot(nblk, m_strict, preferred_element_type=jnp.float32)
        cumb = jnp.dot(nblk, m_incl, preferred_element_type=jnp.float32)
        offp_ref[...] = (offb * BT).astype(jnp.int32)
        nact_ref[...] = jnp.sum(nblk, axis=1, keepdims=True).astype(jnp.int32)
        b_row = lax.broadcasted_iota(jnp.int32, (1, MAXBLK), 1).astype(
            jnp.float32)
        bexp = jnp.zeros((1, MAXBLK), jnp.float32)
        for e in range(EXP):
            bexp = bexp + (b_row >= cumb[:, e:e + 1]).astype(jnp.float32)
        iota8 = lax.broadcasted_iota(jnp.int32, (1, EXP), 1).astype(
            jnp.float32)
        lae = jnp.max(jnp.where(g > 0.5, iota8, 0.0), axis=1, keepdims=True)
        bexp_ref[...] = jnp.minimum(bexp, lae).astype(jnp.int32)


def _route(x, gwt):
    nsteps = T // BT1
    return pl.pallas_call(
        _route_body,
        grid=(nsteps,),
        in_specs=[
            pl.BlockSpec((BT1, D), lambda i: (i, 0)),
            pl.BlockSpec((D, EXP), lambda i: (0, 0)),
        ],
        out_specs=[
            pl.BlockSpec((BT1, EXP), lambda i: (i, 0)),
            pl.BlockSpec((BT1, KTOP), lambda i: (i, 0)),
            pl.BlockSpec((BT1, KTOP), lambda i: (i, 0)),
            pl.BlockSpec((BT1, KTOP), lambda i: (i, 0)),
            pl.BlockSpec((1, EXP), lambda i: (0, 0)),
            pl.BlockSpec((1, MAXBLK), lambda i: (0, 0)),
            pl.BlockSpec((1, 1), lambda i: (0, 0)),
            pl.BlockSpec((BT1, D), lambda i: (i, 0)),
        ],
        out_shape=[
            jax.ShapeDtypeStruct((T, EXP), jnp.float32),
            jax.ShapeDtypeStruct((T, KTOP), jnp.float32),
            jax.ShapeDtypeStruct((T, KTOP), jnp.int32),
            jax.ShapeDtypeStruct((T, KTOP), jnp.int32),
            jax.ShapeDtypeStruct((1, EXP), jnp.int32),
            jax.ShapeDtypeStruct((1, MAXBLK), jnp.int32),
            jax.ShapeDtypeStruct((1, 1), jnp.int32),
            jax.ShapeDtypeStruct((T, D), jnp.bfloat16),
        ],
        scratch_shapes=[pltpu.VMEM((1, EXP), jnp.float32)],
        compiler_params=pltpu.CompilerParams(
            dimension_semantics=("arbitrary",)),
    )(x, gwt)


def _pos_body(topi_ref, rank_ref, offp_ref, pos_ref, p0_ref, p1_ref):
    ti = topi_ref[...]
    acc = rank_ref[...]
    for e in range(EXP):
        acc = acc + jnp.where(ti == e, offp_ref[:, e:e + 1], 0)
    pos_ref[...] = acc
    p0_ref[...] = acc[:, 0:1]
    p1_ref[...] = acc[:, 1:2]


def _pos(topi, rank, offp):
    nsteps = T // BT1
    return pl.pallas_call(
        _pos_body,
        grid=(nsteps,),
        in_specs=[
            pl.BlockSpec((BT1, KTOP), lambda i: (i, 0)),
            pl.BlockSpec((BT1, KTOP), lambda i: (i, 0)),
            pl.BlockSpec((1, EXP), lambda i: (0, 0)),
        ],
        out_specs=[
            pl.BlockSpec((BT1, KTOP), lambda i: (i, 0)),
            pl.BlockSpec((BT1, 1), lambda i: (i, 0)),
            pl.BlockSpec((BT1, 1), lambda i: (i, 0)),
        ],
        out_shape=[
            jax.ShapeDtypeStruct((T, KTOP), jnp.int32),
            jax.ShapeDtypeStruct((T, 1), jnp.int32),
            jax.ShapeDtypeStruct((T, 1), jnp.int32),
        ],
    )(topi, rank, offp)


_PER_W = NA // NW
_CH = 16
_CHD = 64


@functools.cache
def _make_dispatch():
    mesh = plsc.VectorSubcoreMesh(core_axis_name="c", subcore_axis_name="s")

    @functools.partial(
        pl.kernel,
        mesh=mesh,
        out_type=jax.ShapeDtypeStruct((MAXBLK * BT, D // 2), jnp.int32),
        scratch_types=[
            pltpu.VMEM((2, _CHD), jnp.int32),
            pltpu.VMEM((2, _CHD), jnp.int32),
            pltpu.VMEM((_CHD, D // 2), jnp.int32),
            pltpu.VMEM((_CHD, D // 2), jnp.int32),
            pltpu.SemaphoreType.DMA,
            pltpu.SemaphoreType.DMA,
            pltpu.SemaphoreType.DMA,
            pltpu.SemaphoreType.DMA,
        ],
        compiler_params=pltpu.CompilerParams(needs_layout_passes=False),
    )
    def _dispatch(pos_hbm, tok_hbm, x_hbm, xs_hbm, p_v, t_v, rows_a, rows_b,
                  sga, sgb, ssa, ssb):
        wid = lax.axis_index("s") * NC + lax.axis_index("c")
        base = wid * _PER_W
        pltpu.sync_copy(pos_hbm.at[pl.ds(base, _CHD)], p_v.at[0])
        pltpu.sync_copy(tok_hbm.at[pl.ds(base, _CHD)], t_v.at[0])
        pltpu.sync_copy(pos_hbm.at[pl.ds(base + _CHD, _CHD)], p_v.at[1])
        pltpu.sync_copy(tok_hbm.at[pl.ds(base + _CHD, _CHD)], t_v.at[1])
        ga = pltpu.async_copy(x_hbm.at[t_v.at[0]], rows_a, sga)
        gb = pltpu.async_copy(x_hbm.at[t_v.at[1]], rows_b, sgb)
        ga.wait()
        sa = pltpu.async_copy(rows_a, xs_hbm.at[p_v.at[0]], ssa)
        gb.wait()
        sb = pltpu.async_copy(rows_b, xs_hbm.at[p_v.at[1]], ssb)
        sa.wait()
        sb.wait()

    return _dispatch


def _ffn_body(be_ref, na_ref, xs_ref, wg_ref, wu_ref, wd_ref, ys_ref):
    b = pl.program_id(0)

    @pl.when(b < na_ref[0])
    def _():
        x = xs_ref[...].astype(jnp.float32)
        g = jnp.dot(x, wg_ref[0], preferred_element_type=jnp.float32)
        u = jnp.dot(x, wu_ref[0], preferred_element_type=jnp.float32)
        h = g * (1.0 / (1.0 + jnp.exp(-g))) * u
        y = jnp.dot(h, wd_ref[0], preferred_element_type=jnp.float32)
        ys_ref[...] = y.astype(jnp.bfloat16)


def _ffn(bexp, nact, xs, Wg, Wu, Wd):
    grid_spec = pltpu.PrefetchScalarGridSpec(
        num_scalar_prefetch=2,
        grid=(MAXBLK,),
        in_specs=[
            pl.BlockSpec((BT, D),
                         lambda b, be, na: (jnp.minimum(b, na[0] - 1), 0)),
            pl.BlockSpec((1, D, DFF), lambda b, be, na: (be[b], 0, 0)),
            pl.BlockSpec((1, D, DFF), lambda b, be, na: (be[b], 0, 0)),
            pl.BlockSpec((1, DFF, D), lambda b, be, na: (be[b], 0, 0)),
        ],
        out_specs=pl.BlockSpec((BT, D), lambda b, be, na: (b, 0)),
    )
    return pl.pallas_call(
        _ffn_body,
        grid_spec=grid_spec,
        out_shape=jax.ShapeDtypeStruct((MAXBLK * BT, D), jnp.bfloat16),
        compiler_params=pltpu.CompilerParams(
            dimension_semantics=("arbitrary",)),
    )(bexp, nact, xs, Wg, Wu, Wd)


_TPW = T // NW
_CHG = 32


@functools.cache
def _make_gather2():
    mesh = plsc.VectorSubcoreMesh(core_axis_name="c", subcore_axis_name="s")

    @functools.partial(
        pl.kernel,
        mesh=mesh,
        out_type=(jax.ShapeDtypeStruct((T, D // 2), jnp.int32),
                  jax.ShapeDtypeStruct((T, D // 2), jnp.int32)),
        scratch_types=[
            pltpu.VMEM((_CHG,), jnp.int32),
            pltpu.VMEM((_CHG,), jnp.int32),
            pltpu.VMEM((_CHG, D // 2), jnp.int32),
            pltpu.VMEM((_CHG, D // 2), jnp.int32),
            pltpu.SemaphoreType.DMA,
            pltpu.SemaphoreType.DMA,
            pltpu.SemaphoreType.DMA,
            pltpu.SemaphoreType.DMA,
        ],
        compiler_params=pltpu.CompilerParams(needs_layout_passes=False),
    )
    def _gather2(p0_hbm, p1_hbm, ys_hbm, g0_hbm, g1_hbm,
                 pv0, pv1, buf0, buf1, s0, s1, sw0, sw1):
        wid = lax.axis_index("s") * NC + lax.axis_index("c")

        def chunk(c, carry):
            tb = wid * _TPW + c * _CHG
            pltpu.sync_copy(p0_hbm.at[pl.ds(tb, _CHG)], pv0)
            pltpu.sync_copy(p1_hbm.at[pl.ds(tb, _CHG)], pv1)
            cp0 = pltpu.async_copy(ys_hbm.at[pv0], buf0, s0)
            cp1 = pltpu.async_copy(ys_hbm.at[pv1], buf1, s1)
            cp0.wait()
            w0 = pltpu.async_copy(buf0, g0_hbm.at[pl.ds(tb, _CHG)], sw0)
            cp1.wait()
            w1 = pltpu.async_copy(buf1, g1_hbm.at[pl.ds(tb, _CHG)], sw1)
            w0.wait()
            w1.wait()
            return carry

        lax.fori_loop(0, _TPW // _CHG, chunk, 0)

    return _gather2


def _combine_body(g0_ref, g1_ref, w_ref, out_ref):
    w0 = w_ref[:, 0:1]
    w1 = w_ref[:, 1:2]
    out_ref[...] = (g0_ref[...].astype(jnp.float32) * w0
                    + g1_ref[...].astype(jnp.float32) * w1)


def _combine(g0, g1, wts):
    nsteps = T // BT1
    return pl.pallas_call(
        _combine_body,
        grid=(nsteps,),
        in_specs=[
            pl.BlockSpec((BT1, D), lambda i: (i, 0)),
            pl.BlockSpec((BT1, D), lambda i: (i, 0)),
            pl.BlockSpec((BT1, KTOP), lambda i: (i, 0)),
        ],
        out_specs=pl.BlockSpec((BT1, D), lambda i: (i, 0)),
        out_shape=jax.ShapeDtypeStruct((T, D), jnp.float32),
    )(g0, g1, wts)


def kernel(hidden_states, gate_w, Wg, Wu, Wd):
    bsz, seq, dim = hidden_states.shape
    x = hidden_states.reshape(-1, dim)
    logits, wts, topi, rank, offp, bexp, nact, xb = _route(x, gate_w.T)
    pos, p0c, p1c = _pos(topi, rank, offp)
    tok = jnp.repeat(jnp.arange(T, dtype=jnp.int32), KTOP)
    xi = lax.bitcast_convert_type(xb.reshape(T, D // 2, 2), jnp.int32)
    xs_i = _make_dispatch()(pos.reshape(-1), tok, xi)
    xs = lax.bitcast_convert_type(xs_i, jnp.bfloat16).reshape(MAXBLK * BT, D)
    ys = _ffn(bexp.reshape(-1), nact.reshape(-1), xs, Wg, Wu, Wd)
    ys_i = lax.bitcast_convert_type(
        ys.reshape(MAXBLK * BT, D // 2, 2), jnp.int32)
    g0_i, g1_i = _make_gather2()(p0c.reshape(-1), p1c.reshape(-1), ys_i)
    g0 = lax.bitcast_convert_type(g0_i, jnp.bfloat16).reshape(T, D)
    g1 = lax.bitcast_convert_type(g1_i, jnp.bfloat16).reshape(T, D)
    out = _combine(g0, g1, wts)
    return out.reshape(bsz, seq, dim), logits


# speedup vs baseline: 3.5958x; 3.5958x over previous
import functools

import jax
import jax.numpy as jnp
from jax import lax
from jax.experimental import pallas as pl
from jax.experimental.pallas import tpu as pltpu
from jax.experimental.pallas import tpu_sc as plsc

T = 2048
D = 1024
DFF = 1024
EXP = 8
KTOP = 2
NA = T * KTOP

BT = 256
MAXBLK = NA // BT + EXP - 1
BT1 = 512

NC = 2
NS = 16
NW = NC * NS


def _route_body(x_ref, gwt_ref, logits_ref, wts_ref, offp_ref, bexp_ref,
                nact_ref, pos_ref, p0_ref, p1_ref, carry, ti_s, rk_s):
    i = pl.program_id(0)
    nsteps = T // BT1

    @pl.when(i == 0)
    def _init():
        carry[...] = jnp.zeros_like(carry)

    @pl.when(i == nsteps)
    def _pos_step():
        ti = ti_s[...]
        acc = rk_s[...]
        offv = offp_ref[...]
        for e in range(EXP):
            acc = acc + jnp.where(ti == e, offv[:, e:e + 1], 0)
        pos_ref[...] = acc
        p0_ref[...] = acc[:, 0:1]
        p1_ref[...] = acc[:, 1:2]

    @pl.when(i < nsteps)
    def _route_step():
        _route_step_body(i, nsteps, x_ref, gwt_ref, logits_ref, wts_ref,
                         offp_ref, bexp_ref, nact_ref, carry, ti_s, rk_s)


def _route_step_body(i, nsteps, x_ref, gwt_ref, logits_ref, wts_ref,
                     offp_ref, bexp_ref, nact_ref, carry, ti_s, rk_s):
    x = x_ref[...]
    logits = jnp.dot(x, gwt_ref[...], preferred_element_type=jnp.float32)
    logits_ref[...] = logits

    m = jnp.max(logits, axis=1, keepdims=True)
    ex = jnp.exp(logits - m)
    p = ex / jnp.sum(ex, axis=1, keepdims=True)

    iota_e = lax.broadcasted_iota(jnp.int32, (BT1, EXP), 1)
    v0 = jnp.max(p, axis=1, keepdims=True)
    i0 = jnp.min(jnp.where(p == v0, iota_e, EXP), axis=1, keepdims=True)
    pm = jnp.where(iota_e == i0, -jnp.inf, p)
    v1 = jnp.max(pm, axis=1, keepdims=True)
    i1 = jnp.min(jnp.where(pm == v1, iota_e, EXP), axis=1, keepdims=True)
    s = v0 + v1
    wts_ref[...] = jnp.concatenate([v0 / s, v1 / s], axis=1)
    ti_s[pl.ds(i * BT1, BT1), :] = jnp.concatenate([i0, i1], axis=1)

    oh0 = (iota_e == i0).astype(jnp.float32)
    oh1 = (iota_e == i1).astype(jnp.float32)
    r_i = lax.broadcasted_iota(jnp.int32, (BT1, BT1), 0)
    c_i = lax.broadcasted_iota(jnp.int32, (BT1, BT1), 1)
    ltri = (c_i < r_i).astype(jnp.float32)
    cums = jnp.dot(ltri, jnp.concatenate([oh0, oh1], axis=1),
                   preferred_element_type=jnp.float32)
    cums0 = cums[:, :EXP]
    cums1 = cums[:, EXP:]
    base = carry[...]
    r0 = jnp.sum(oh0 * (base + cums0 + cums1), axis=1, keepdims=True)
    r1 = jnp.sum(oh1 * (base + cums0 + oh0 + cums1), axis=1, keepdims=True)
    rk_s[pl.ds(i * BT1, BT1), :] = jnp.concatenate(
        [r0, r1], axis=1).astype(jnp.int32)
    newc = base + jnp.sum(oh0 + oh1, axis=0, keepdims=True)
    carry[...] = newc

    @pl.when(i == nsteps - 1)
    def _epilogue():
        g = newc
        nblk = jnp.floor((g + (BT - 1)) * (1.0 / BT))
        e_r = lax.broadcasted_iota(jnp.int32, (EXP, EXP), 0)
        e_c = lax.broadcasted_iota(jnp.int32, (EXP, EXP), 1)
        m_strict = (e_r < e_c).astype(jnp.float32)
        m_incl = (e_r <= e_c).astype(jnp.float32)
        offb = jnp.dot(nblk, m_strict, preferred_element_type=jnp.float32)
        cumb = jnp.dot(nblk, m_incl, preferred_element_type=jnp.float32)
        offp_ref[...] = (offb * BT).astype(jnp.int32)
        nact_ref[...] = jnp.sum(nblk, axis=1, keepdims=True).astype(jnp.int32)
        b_row = lax.broadcasted_iota(jnp.int32, (1, MAXBLK), 1).astype(
            jnp.float32)
        bexp = jnp.zeros((1, MAXBLK), jnp.float32)
        for e in range(EXP):
            bexp = bexp + (b_row >= cumb[:, e:e + 1]).astype(jnp.float32)
        iota8 = lax.broadcasted_iota(jnp.int32, (1, EXP), 1).astype(
            jnp.float32)
        lae = jnp.max(jnp.where(g > 0.5, iota8, 0.0), axis=1, keepdims=True)
        bexp_ref[...] = jnp.minimum(bexp, lae).astype(jnp.int32)


def _route(x, gwt):
    nsteps = T // BT1
    return pl.pallas_call(
        _route_body,
        grid=(nsteps + 1,),
        in_specs=[
            pl.BlockSpec((BT1, D), lambda i: (jnp.minimum(i, nsteps - 1), 0)),
            pl.BlockSpec((D, EXP), lambda i: (0, 0)),
        ],
        out_specs=[
            pl.BlockSpec((BT1, EXP),
                         lambda i: (jnp.minimum(i, nsteps - 1), 0)),
            pl.BlockSpec((BT1, KTOP),
                         lambda i: (jnp.minimum(i, nsteps - 1), 0)),
            pl.BlockSpec((1, EXP), lambda i: (0, 0)),
            pl.BlockSpec((1, MAXBLK), lambda i: (0, 0)),
            pl.BlockSpec((1, 1), lambda i: (0, 0)),
            pl.BlockSpec((T, KTOP), lambda i: (0, 0)),
            pl.BlockSpec((T, 1), lambda i: (0, 0)),
            pl.BlockSpec((T, 1), lambda i: (0, 0)),
        ],
        out_shape=[
            jax.ShapeDtypeStruct((T, EXP), jnp.float32),
            jax.ShapeDtypeStruct((T, KTOP), jnp.float32),
            jax.ShapeDtypeStruct((1, EXP), jnp.int32),
            jax.ShapeDtypeStruct((1, MAXBLK), jnp.int32),
            jax.ShapeDtypeStruct((1, 1), jnp.int32),
            jax.ShapeDtypeStruct((T, KTOP), jnp.int32),
            jax.ShapeDtypeStruct((T, 1), jnp.int32),
            jax.ShapeDtypeStruct((T, 1), jnp.int32),
        ],
        scratch_shapes=[pltpu.VMEM((1, EXP), jnp.float32),
                        pltpu.VMEM((T, KTOP), jnp.int32),
                        pltpu.VMEM((T, KTOP), jnp.int32)],
        compiler_params=pltpu.CompilerParams(
            dimension_semantics=("arbitrary",)),
    )(x, gwt)


_PER_W = NA // NW
_CHD = 32
_CHG = 32


@functools.cache
def _make_dispatch():
    mesh = plsc.VectorSubcoreMesh(core_axis_name="c", subcore_axis_name="s")

    @functools.partial(
        pl.kernel,
        mesh=mesh,
        out_type=jax.ShapeDtypeStruct((MAXBLK * BT, D), jnp.float32),
        scratch_types=[
            pltpu.VMEM((4, _CHD), jnp.int32),
            pltpu.VMEM((4, _CHD), jnp.int32),
            pltpu.VMEM((_CHD, D), jnp.float32),
            pltpu.VMEM((_CHD, D), jnp.float32),
            pltpu.SemaphoreType.DMA,
            pltpu.SemaphoreType.DMA,
            pltpu.SemaphoreType.DMA,
            pltpu.SemaphoreType.DMA,
        ],
        compiler_params=pltpu.CompilerParams(needs_layout_passes=False),
    )
    def _dispatch(pos_hbm, tok_hbm, x_hbm, xs_hbm, p_v, t_v, rows_a, rows_b,
                  sga, sgb, ssa, ssb):
        wid = lax.axis_index("s") * NC + lax.axis_index("c")
        base = wid * _PER_W
        for c in range(4):
            pltpu.sync_copy(pos_hbm.at[pl.ds(base + c * _CHD, _CHD)],
                            p_v.at[c])
            pltpu.sync_copy(tok_hbm.at[pl.ds(base + c * _CHD, _CHD)],
                            t_v.at[c])
        ga = pltpu.async_copy(x_hbm.at[t_v.at[0]], rows_a, sga)
        gb = pltpu.async_copy(x_hbm.at[t_v.at[1]], rows_b, sgb)
        ga.wait()
        sa = pltpu.async_copy(rows_a, xs_hbm.at[p_v.at[0]], ssa)
        gb.wait()
        sb = pltpu.async_copy(rows_b, xs_hbm.at[p_v.at[1]], ssb)
        sa.wait()
        ga2 = pltpu.async_copy(x_hbm.at[t_v.at[2]], rows_a, sga)
        sb.wait()
        gb2 = pltpu.async_copy(x_hbm.at[t_v.at[3]], rows_b, sgb)
        ga2.wait()
        sa2 = pltpu.async_copy(rows_a, xs_hbm.at[p_v.at[2]], ssa)
        gb2.wait()
        sb2 = pltpu.async_copy(rows_b, xs_hbm.at[p_v.at[3]], ssb)
        sa2.wait()
        sb2.wait()

    return _dispatch


def _ffn_body(be_ref, na_ref, xs_ref, wg_ref, wu_ref, wd_ref, ys_ref):
    b = pl.program_id(0)

    @pl.when(b < na_ref[0])
    def _():
        x = xs_ref[...]
        g = jnp.dot(x, wg_ref[0], preferred_element_type=jnp.float32)
        u = jnp.dot(x, wu_ref[0], preferred_element_type=jnp.float32)
        h = g * (1.0 / (1.0 + jnp.exp(-g))) * u
        ys_ref[...] = jnp.dot(h, wd_ref[0], preferred_element_type=jnp.float32)


def _ffn(bexp, nact, xs, Wg, Wu, Wd):
    grid_spec = pltpu.PrefetchScalarGridSpec(
        num_scalar_prefetch=2,
        grid=(MAXBLK,),
        in_specs=[
            pl.BlockSpec((BT, D),
                         lambda b, be, na: (jnp.minimum(b, na[0] - 1), 0)),
            pl.BlockSpec((1, D, DFF), lambda b, be, na: (be[b], 0, 0)),
            pl.BlockSpec((1, D, DFF), lambda b, be, na: (be[b], 0, 0)),
            pl.BlockSpec((1, DFF, D), lambda b, be, na: (be[b], 0, 0)),
        ],
        out_specs=pl.BlockSpec((BT, D), lambda b, be, na: (b, 0)),
    )
    return pl.pallas_call(
        _ffn_body,
        grid_spec=grid_spec,
        out_shape=jax.ShapeDtypeStruct((MAXBLK * BT, D), jnp.float32),
        compiler_params=pltpu.CompilerParams(
            dimension_semantics=("arbitrary",)),
    )(bexp, nact, xs, Wg, Wu, Wd)


_TPW = T // NW


@functools.cache
def _make_gather2():
    mesh = plsc.VectorSubcoreMesh(core_axis_name="c", subcore_axis_name="s")

    @functools.partial(
        pl.kernel,
        mesh=mesh,
        out_type=(jax.ShapeDtypeStruct((T, D), jnp.float32),
                  jax.ShapeDtypeStruct((T, D), jnp.float32)),
        scratch_types=[
            pltpu.VMEM((_CHG,), jnp.int32),
            pltpu.VMEM((_CHG,), jnp.int32),
            pltpu.VMEM((_CHG, D), jnp.float32),
            pltpu.VMEM((_CHG, D), jnp.float32),
            pltpu.SemaphoreType.DMA,
            pltpu.SemaphoreType.DMA,
            pltpu.SemaphoreType.DMA,
            pltpu.SemaphoreType.DMA,
        ],
        compiler_params=pltpu.CompilerParams(needs_layout_passes=False),
    )
    def _gather2(p0_hbm, p1_hbm, ys_hbm, g0_hbm, g1_hbm,
                 pv0, pv1, buf0, buf1, s0, s1, sw0, sw1):
        wid = lax.axis_index("s") * NC + lax.axis_index("c")

        def chunk(c, carry):
            tb = wid * _TPW + c * _CHG
            pltpu.sync_copy(p0_hbm.at[pl.ds(tb, _CHG)], pv0)
            pltpu.sync_copy(p1_hbm.at[pl.ds(tb, _CHG)], pv1)
            cp0 = pltpu.async_copy(ys_hbm.at[pv0], buf0, s0)
            cp1 = pltpu.async_copy(ys_hbm.at[pv1], buf1, s1)
            cp0.wait()
            w0 = pltpu.async_copy(buf0, g0_hbm.at[pl.ds(tb, _CHG)], sw0)
            cp1.wait()
            w1 = pltpu.async_copy(buf1, g1_hbm.at[pl.ds(tb, _CHG)], sw1)
            w0.wait()
            w1.wait()
            return carry

        lax.fori_loop(0, _TPW // _CHG, chunk, 0)

    return _gather2


def _combine_body(g0_ref, g1_ref, w_ref, out_ref):
    w0 = w_ref[:, 0:1]
    w1 = w_ref[:, 1:2]
    out_ref[...] = (g0_ref[...].astype(jnp.float32) * w0
                    + g1_ref[...].astype(jnp.float32) * w1)


def _combine(g0, g1, wts):
    nsteps = T // BT1
    return pl.pallas_call(
        _combine_body,
        grid=(nsteps,),
        in_specs=[
            pl.BlockSpec((BT1, D), lambda i: (i, 0)),
            pl.BlockSpec((BT1, D), lambda i: (i, 0)),
            pl.BlockSpec((BT1, KTOP), lambda i: (i, 0)),
        ],
        out_specs=pl.BlockSpec((BT1, D), lambda i: (i, 0)),
        out_shape=jax.ShapeDtypeStruct((T, D), jnp.float32),
    )(g0, g1, wts)


def kernel(hidden_states, gate_w, Wg, Wu, Wd):
    bsz, seq, dim = hidden_states.shape
    x = hidden_states.reshape(-1, dim)
    (logits, wts, offp, bexp, nact,
     pos, p0c, p1c) = _route(x, gate_w.T)
    tok = jnp.repeat(jnp.arange(T, dtype=jnp.int32), KTOP)
    xs = _make_dispatch()(pos.reshape(-1), tok, x)
    ys = _ffn(bexp.reshape(-1), nact.reshape(-1), xs, Wg, Wu, Wd)
    g0, g1 = _make_gather2()(p0c.reshape(-1), p1c.reshape(-1), ys)
    out = _combine(g0, g1, wts)
    return out.reshape(bsz, seq, dim), logits
